# baseline (speedup 1.0000x reference)
import functools

import jax
import jax.numpy as jnp
from jax import lax
from jax.experimental import pallas as pl
from jax.experimental.pallas import tpu as pltpu
from jax.experimental.pallas import tpu_sc as plsc

B, N, H, D = 128, 64, 32, 768
NUM_HOPS = 5
ATOM_VOCAB = 512 * 9 + 1
EDGE_VOCAB = 512 * 3 + 1
DEG_VOCAB = 512
NPAT = 1 << NUM_HOPS

NPOS = N * N
NCHUNK = 8
CHUNK = NPOS // NCHUNK

NC, NS = 2, 16
NW = NC * NS

BB = 8


def _apow_body(adj_ref, pat_ref, explored_ref):
    a32 = adj_ref[...]
    a16 = a32.astype(jnp.bfloat16)
    ak = a16
    acc = a32
    pat = jnp.zeros_like(a32)
    for hop in range(NUM_HOPS):
        pat = pat + float(1 << hop) * ak.astype(jnp.float32)
        prod = lax.dot_general(
            ak, a16,
            dimension_numbers=(((2,), (1,)), ((0,), (0,))),
            preferred_element_type=jnp.float32)
        akn = jnp.minimum(prod, 1.0)
        acc = acc + akn
        ak = akn.astype(jnp.bfloat16)
    pat_ref[...] = pat.astype(jnp.int32)
    explored_ref[...] = (acc > 0).astype(jnp.float32)


_apow_call = pl.pallas_call(
    _apow_body,
    grid=(B // BB,),
    in_specs=[pl.BlockSpec((BB, N, N), lambda i: (i, 0, 0))],
    out_specs=[
        pl.BlockSpec((BB, N, N), lambda i: (i, 0, 0)),
        pl.BlockSpec((BB, N, N), lambda i: (i, 0, 0)),
    ],
    out_shape=[
        jax.ShapeDtypeStruct((B, N, N), jnp.int32),
        jax.ShapeDtypeStruct((B, N, N), jnp.float32),
    ],
)

PAIRS = B * N
PPW = PAIRS // NW
CP = 4
ROWS = CP * 10
NCHN = PPW // CP

_sc_mesh = plsc.VectorSubcoreMesh(core_axis_name="c", subcore_axis_name="s")


@functools.partial(
    pl.kernel,
    mesh=_sc_mesh,
    out_type=jax.ShapeDtypeStruct((PAIRS, D), jnp.float32),
    compiler_params=pltpu.CompilerParams(
        needs_layout_passes=False, use_tc_tiling_on_sc=False),
    scratch_types=[
        pltpu.VMEM((NCHN, ROWS), jnp.int32),
        pltpu.VMEM((ROWS, D), jnp.float32),
        pltpu.VMEM((CP, D), jnp.float32),
        pltpu.SemaphoreType.DMA,
    ],
)
def _node_gather(table_hbm, idx_hbm, out_hbm, idx_v, rows_v, out_v, sem):
    wid = lax.axis_index("s") * NC + lax.axis_index("c")
    pltpu.sync_copy(idx_hbm.at[wid], idx_v)

    def chunk(c, carry):
        pltpu.async_copy(table_hbm.at[idx_v.at[c]], rows_v, sem).wait()
        for p in range(CP):
            def dloop(j, carry2):
                sl = pl.ds(j * 16, 16)
                acc = rows_v[p * 10, sl]
                for k in range(1, 10):
                    acc = acc + rows_v[p * 10 + k, sl]
                out_v[p, sl] = acc
                return carry2
            lax.fori_loop(0, D // 16, dloop, 0)
        pltpu.sync_copy(out_v, out_hbm.at[pl.ds(wid * PPW + c * CP, CP), :])
        return carry

    lax.fori_loop(0, NCHN, chunk, 0)


BPW = B // NW


@functools.partial(
    pl.kernel,
    mesh=_sc_mesh,
    out_type=jax.ShapeDtypeStruct((B, H, NPOS), jnp.float32),
    compiler_params=pltpu.CompilerParams(
        needs_layout_passes=False, use_tc_tiling_on_sc=False),
    scratch_types=[
        pltpu.VMEM((EDGE_VOCAB, H), jnp.float32),
        pltpu.VMEM((CHUNK, 3), jnp.int32),
        pltpu.VMEM((CHUNK,), jnp.int32),
        pltpu.VMEM((H, CHUNK), jnp.float32),
        pltpu.VMEM((H, CHUNK), jnp.float32),
        pltpu.VMEM((NPAT, H), jnp.float32),
        pltpu.SemaphoreType.DMA,
    ],
)
def _edge_merge(tab_hbm, eidx_hbm, pos_hbm, pat_hbm, faclut_hbm, att_hbm,
                tab_v, eidx_v, pat_v, pos_v, out_v, faclut_v, sem):
    wid = lax.axis_index("s") * NC + lax.axis_index("c")
    pltpu.sync_copy(tab_hbm, tab_v)
    pltpu.sync_copy(faclut_hbm, faclut_v)

    def body(t, carry):
        b = wid * BPW + t // NCHUNK
        c = t % NCHUNK
        pltpu.sync_copy(eidx_hbm.at[b, pl.ds(c * CHUNK, CHUNK), :], eidx_v)
        pltpu.sync_copy(pat_hbm.at[b, pl.ds(c * CHUNK, CHUNK)], pat_v)
        pltpu.sync_copy(pos_hbm.at[b, :, pl.ds(c * CHUNK, CHUNK)], pos_v)

        @functools.partial(plsc.parallel_loop, 0, CHUNK // 16, unroll=2)
        def group(g):
            sl = pl.ds(g * 16, 16)
            rows = lax.iota(jnp.int32, 16) + g * 16
            e0 = plsc.load_gather(eidx_v, [rows, jnp.zeros((16,), jnp.int32)])
            e1 = plsc.load_gather(eidx_v, [rows, jnp.ones((16,), jnp.int32)])
            e2 = plsc.load_gather(eidx_v, [rows, jnp.full((16,), 2, jnp.int32)])
            pat = pat_v[sl]
            for h in range(H):
                hsplat = jnp.full((16,), h, jnp.int32)
                fac = plsc.load_gather(faclut_v, [pat, hsplat])
                g0 = plsc.load_gather(tab_v, [e0, hsplat])
                g1 = plsc.load_gather(tab_v, [e1, hsplat])
                g2 = plsc.load_gather(tab_v, [e2, hsplat])
                out_v[h, sl] = (pos_v[h, sl] + g0 + g1 + g2) * fac

        pltpu.sync_copy(out_v, att_hbm.at[b, :, pl.ds(c * CHUNK, CHUNK)])
        return carry

    lax.fori_loop(0, BPW * NCHUNK, body, 0)


def kernel(node_feat_idx, degree, edge_feat_idx, adj, position_bias,
           atom_table, edge_table, degree_table, node_vnode,
           node_vnode_distance, diffusion_weight):
    combined = jnp.concatenate([atom_table, degree_table], axis=0)
    idx_all = jnp.concatenate(
        [node_feat_idx, degree[..., None] + ATOM_VOCAB], axis=-1)
    idx_node = idx_all.astype(jnp.int32).reshape(NW, NCHN, ROWS)
    node_features = _node_gather(combined, idx_node).reshape(B, N, D)

    pat, explored = _apow_call(adj)
    pat2 = pat.reshape(B, NPOS)
    bits = ((jnp.arange(NPAT)[:, None] >> jnp.arange(NUM_HOPS)[None, :])
            & 1).astype(jnp.float32)
    faclut = 1.0 + bits @ diffusion_weight
    eidx3 = edge_feat_idx.astype(jnp.int32).reshape(B, NPOS, 3)
    pos3 = position_bias.reshape(B, H, NPOS)
    att3 = _edge_merge(edge_table, eidx3, pos3, pat2, faclut)
    att_bias = att3.reshape(B, H, N, N)
    return (node_features, att_bias, explored, node_vnode,
            node_vnode_distance)


# speedup vs baseline: 1.2736x; 1.2736x over previous
import functools

import jax
import jax.numpy as jnp
from jax import lax
from jax.experimental import pallas as pl
from jax.experimental.pallas import tpu as pltpu
from jax.experimental.pallas import tpu_sc as plsc

B, N, H, D = 128, 64, 32, 768
NUM_HOPS = 5
ATOM_VOCAB = 512 * 9 + 1
EDGE_VOCAB = 512 * 3 + 1
DEG_VOCAB = 512
NPAT = 1 << NUM_HOPS

NPOS = N * N
NCHUNK = 8
CHUNK = NPOS // NCHUNK

NC, NS = 2, 16
NW = NC * NS

BB = 8


def _apow_body(adj_ref, pat_ref, explored_ref):
    a32 = adj_ref[...]
    a16 = a32.astype(jnp.bfloat16)
    ak = a16
    acc = a32
    pat = jnp.zeros_like(a32)
    for hop in range(NUM_HOPS):
        pat = pat + float(1 << hop) * ak.astype(jnp.float32)
        prod = lax.dot_general(
            ak, a16,
            dimension_numbers=(((2,), (1,)), ((0,), (0,))),
            preferred_element_type=jnp.float32)
        akn = jnp.minimum(prod, 1.0)
        acc = acc + akn
        ak = akn.astype(jnp.bfloat16)
    pat_ref[...] = pat.astype(jnp.int32)
    explored_ref[...] = (acc > 0).astype(jnp.float32)


_apow_call = pl.pallas_call(
    _apow_body,
    grid=(B // BB,),
    in_specs=[pl.BlockSpec((BB, N, N), lambda i: (i, 0, 0))],
    out_specs=[
        pl.BlockSpec((BB, N, N), lambda i: (i, 0, 0)),
        pl.BlockSpec((BB, N, N), lambda i: (i, 0, 0)),
    ],
    out_shape=[
        jax.ShapeDtypeStruct((B, N, N), jnp.int32),
        jax.ShapeDtypeStruct((B, N, N), jnp.float32),
    ],
)

PAIRS = B * N
PPW = PAIRS // NW
CP = 4
ROWS = CP * 10
NCHN = PPW // CP

_sc_mesh = plsc.VectorSubcoreMesh(core_axis_name="c", subcore_axis_name="s")


@functools.partial(
    pl.kernel,
    mesh=_sc_mesh,
    out_type=jax.ShapeDtypeStruct((PAIRS, D), jnp.float32),
    compiler_params=pltpu.CompilerParams(
        needs_layout_passes=False, use_tc_tiling_on_sc=False),
    scratch_types=[
        pltpu.VMEM((NCHN, ROWS), jnp.int32),
        pltpu.VMEM((ROWS, D), jnp.float32),
        pltpu.VMEM((CP, D), jnp.float32),
        pltpu.SemaphoreType.DMA,
    ],
)
def _node_gather(table_hbm, idx_hbm, out_hbm, idx_v, rows_v, out_v, sem):
    wid = lax.axis_index("s") * NC + lax.axis_index("c")
    pltpu.sync_copy(idx_hbm.at[wid], idx_v)

    def chunk(c, carry):
        pltpu.async_copy(table_hbm.at[idx_v.at[c]], rows_v, sem).wait()
        for p in range(CP):
            def dloop(j, carry2):
                sl = pl.ds(j * 16, 16)
                acc = rows_v[p * 10, sl]
                for k in range(1, 10):
                    acc = acc + rows_v[p * 10 + k, sl]
                out_v[p, sl] = acc
                return carry2
            lax.fori_loop(0, D // 16, dloop, 0)
        pltpu.sync_copy(out_v, out_hbm.at[pl.ds(wid * PPW + c * CP, CP), :])
        return carry

    lax.fori_loop(0, NCHN, chunk, 0)


BPW = B // NW


@functools.partial(
    pl.kernel,
    mesh=_sc_mesh,
    out_type=jax.ShapeDtypeStruct((B, H, NPOS), jnp.float32),
    compiler_params=pltpu.CompilerParams(
        needs_layout_passes=False, use_tc_tiling_on_sc=False),
    scratch_types=[
        pltpu.VMEM((EDGE_VOCAB, H), jnp.float32),
        pltpu.VMEM((3, CHUNK), jnp.int32),
        pltpu.VMEM((CHUNK,), jnp.int32),
        pltpu.VMEM((H, CHUNK), jnp.float32),
        pltpu.VMEM((H, CHUNK), jnp.float32),
        pltpu.VMEM((NPAT, H), jnp.float32),
        pltpu.SemaphoreType.DMA,
    ],
)
def _edge_merge(tab_hbm, eidx_hbm, pos_hbm, pat_hbm, faclut_hbm, att_hbm,
                tab_v, eidx_v, pat_v, pos_v, out_v, faclut_v, sem):
    wid = lax.axis_index("s") * NC + lax.axis_index("c")
    pltpu.sync_copy(tab_hbm, tab_v)
    pltpu.sync_copy(faclut_hbm, faclut_v)

    def body(t, carry):
        b = wid * BPW + t // NCHUNK
        c = t % NCHUNK
        pltpu.sync_copy(eidx_hbm.at[b, c], eidx_v)
        pltpu.sync_copy(pat_hbm.at[b, pl.ds(c * CHUNK, CHUNK)], pat_v)
        pltpu.sync_copy(pos_hbm.at[b, :, pl.ds(c * CHUNK, CHUNK)], pos_v)

        def group(g, carry2):
            sl = pl.ds(g * 16, 16)
            e0 = eidx_v[0, sl]
            e1 = eidx_v[1, sl]
            e2 = eidx_v[2, sl]
            pat = pat_v[sl]
            for h in range(H):
                hsplat = jnp.full((16,), h, jnp.int32)
                fac = plsc.load_gather(faclut_v, [pat, hsplat])
                g0 = plsc.load_gather(tab_v, [e0, hsplat])
                g1 = plsc.load_gather(tab_v, [e1, hsplat])
                g2 = plsc.load_gather(tab_v, [e2, hsplat])
                out_v[h, sl] = (pos_v[h, sl] + g0 + g1 + g2) * fac
            return carry2

        lax.fori_loop(0, CHUNK // 16, group, 0)

        pltpu.sync_copy(out_v, att_hbm.at[b, :, pl.ds(c * CHUNK, CHUNK)])
        return carry

    lax.fori_loop(0, BPW * NCHUNK, body, 0)


def kernel(node_feat_idx, degree, edge_feat_idx, adj, position_bias,
           atom_table, edge_table, degree_table, node_vnode,
           node_vnode_distance, diffusion_weight):
    combined = jnp.concatenate([atom_table, degree_table], axis=0)
    idx_all = jnp.concatenate(
        [node_feat_idx, degree[..., None] + ATOM_VOCAB], axis=-1)
    idx_node = idx_all.astype(jnp.int32).reshape(NW, NCHN, ROWS)
    node_features = _node_gather(combined, idx_node).reshape(B, N, D)

    pat, explored = _apow_call(adj)
    pat2 = pat.reshape(B, NPOS)
    bits = ((jnp.arange(NPAT)[:, None] >> jnp.arange(NUM_HOPS)[None, :])
            & 1).astype(jnp.float32)
    faclut = 1.0 + bits @ diffusion_weight
    eidx4 = edge_feat_idx.astype(jnp.int32).reshape(
        B, NCHUNK, CHUNK, 3).transpose(0, 1, 3, 2)
    pos3 = position_bias.reshape(B, H, NPOS)
    att3 = _edge_merge(edge_table, eidx4, pos3, pat2, faclut)
    att_bias = att3.reshape(B, H, N, N)
    return (node_features, att_bias, explored, node_vnode,
            node_vnode_distance)


# speedup vs baseline: 1.6774x; 1.3170x over previous
import functools

import jax
import jax.numpy as jnp
from jax import lax
from jax.experimental import pallas as pl
from jax.experimental.pallas import tpu as pltpu
from jax.experimental.pallas import tpu_sc as plsc

B, N, H, D = 128, 64, 32, 768
NUM_HOPS = 5
ATOM_VOCAB = 512 * 9 + 1
EDGE_VOCAB = 512 * 3 + 1
DEG_VOCAB = 512
NPAT = 1 << NUM_HOPS

NPOS = N * N
NCHUNK = 8
CHUNK = NPOS // NCHUNK

NC, NS = 2, 16
NW = NC * NS

BB = 8


def _apow_body(adj_ref, pat_ref, explored_ref):
    a32 = adj_ref[...]
    a16 = a32.astype(jnp.bfloat16)
    ak = a16
    acc = a32
    pat = jnp.zeros_like(a32)
    for hop in range(NUM_HOPS):
        pat = pat + float(1 << hop) * ak.astype(jnp.float32)
        prod = lax.dot_general(
            ak, a16,
            dimension_numbers=(((2,), (1,)), ((0,), (0,))),
            preferred_element_type=jnp.float32)
        akn = jnp.minimum(prod, 1.0)
        acc = acc + akn
        ak = akn.astype(jnp.bfloat16)
    pat_ref[...] = pat.astype(jnp.int32)
    explored_ref[...] = (acc > 0).astype(jnp.float32)


_apow_call = pl.pallas_call(
    _apow_body,
    grid=(B // BB,),
    in_specs=[pl.BlockSpec((BB, N, N), lambda i: (i, 0, 0))],
    out_specs=[
        pl.BlockSpec((BB, N, N), lambda i: (i, 0, 0)),
        pl.BlockSpec((BB, N, N), lambda i: (i, 0, 0)),
    ],
    out_shape=[
        jax.ShapeDtypeStruct((B, N, N), jnp.int32),
        jax.ShapeDtypeStruct((B, N, N), jnp.float32),
    ],
)

PAIRS = B * N
PPW = PAIRS // NW
CP = 4
ROWS = CP * 10
NCHN = PPW // CP

_sc_mesh = plsc.VectorSubcoreMesh(core_axis_name="c", subcore_axis_name="s")


@functools.partial(
    pl.kernel,
    mesh=_sc_mesh,
    out_type=jax.ShapeDtypeStruct((PAIRS, D), jnp.float32),
    compiler_params=pltpu.CompilerParams(
        needs_layout_passes=False, use_tc_tiling_on_sc=False),
    scratch_types=[
        pltpu.VMEM((NCHN, ROWS), jnp.int32),
        pltpu.VMEM((ROWS, D), jnp.float32),
        pltpu.VMEM((CP, D), jnp.float32),
        pltpu.SemaphoreType.DMA,
    ],
)
def _node_gather(table_hbm, idx_hbm, out_hbm, idx_v, rows_v, out_v, sem):
    wid = lax.axis_index("s") * NC + lax.axis_index("c")
    pltpu.sync_copy(idx_hbm.at[wid], idx_v)

    def chunk(c, carry):
        pltpu.async_copy(table_hbm.at[idx_v.at[c]], rows_v, sem).wait()
        for p in range(CP):
            def dloop(j, carry2):
                sl = pl.ds(j * 16, 16)
                acc = rows_v[p * 10, sl]
                for k in range(1, 10):
                    acc = acc + rows_v[p * 10 + k, sl]
                out_v[p, sl] = acc
                return carry2
            lax.fori_loop(0, D // 16, dloop, 0)
        pltpu.sync_copy(out_v, out_hbm.at[pl.ds(wid * PPW + c * CP, CP), :])
        return carry

    lax.fori_loop(0, NCHN, chunk, 0)


BPW = B // NW


@functools.partial(
    pl.kernel,
    mesh=_sc_mesh,
    out_type=jax.ShapeDtypeStruct((B, H, NPOS), jnp.float32),
    compiler_params=pltpu.CompilerParams(
        needs_layout_passes=False, use_tc_tiling_on_sc=False),
    scratch_types=[
        pltpu.VMEM((EDGE_VOCAB, H), jnp.float32),
        pltpu.VMEM((3, CHUNK), jnp.int32),
        pltpu.VMEM((CHUNK,), jnp.int32),
        pltpu.VMEM((H, CHUNK), jnp.float32),
        pltpu.VMEM((H, CHUNK), jnp.float32),
        pltpu.VMEM((NPAT, H), jnp.float32),
        pltpu.SemaphoreType.DMA,
    ],
)
def _edge_merge(tab_hbm, eidx_hbm, pos_hbm, pat_hbm, faclut_hbm, att_hbm,
                tab_v, eidx_v, pat_v, pos_v, out_v, faclut_v, sem):
    wid = lax.axis_index("s") * NC + lax.axis_index("c")
    pltpu.sync_copy(tab_hbm, tab_v)
    pltpu.sync_copy(faclut_hbm, faclut_v)

    def body(t, carry):
        b = wid * BPW + t // NCHUNK
        c = t % NCHUNK
        pltpu.sync_copy(eidx_hbm.at[b, c], eidx_v)
        pltpu.sync_copy(pat_hbm.at[b, pl.ds(c * CHUNK, CHUNK)], pat_v)
        pltpu.sync_copy(pos_hbm.at[b, :, pl.ds(c * CHUNK, CHUNK)], pos_v)

        def group(g, carry2):
            sl = pl.ds(g * 16, 16)
            e0 = eidx_v[0, sl]
            e1 = eidx_v[1, sl]
            e2 = eidx_v[2, sl]
            pat = pat_v[sl]
            HB = 8
            for hb in range(0, H, HB):
                hs = [jnp.full((16,), hb + i, jnp.int32) for i in range(HB)]
                fs = [plsc.load_gather(faclut_v, [pat, hs[i]])
                      for i in range(HB)]
                p_s = [pos_v[hb + i, sl] for i in range(HB)]
                g0s = [plsc.load_gather(tab_v, [e0, hs[i]]) for i in range(HB)]
                g1s = [plsc.load_gather(tab_v, [e1, hs[i]]) for i in range(HB)]
                g2s = [plsc.load_gather(tab_v, [e2, hs[i]]) for i in range(HB)]
                for i in range(HB):
                    out_v[hb + i, sl] = (
                        (p_s[i] + g0s[i] + g1s[i] + g2s[i]) * fs[i])
            return carry2

        lax.fori_loop(0, CHUNK // 16, group, 0)

        pltpu.sync_copy(out_v, att_hbm.at[b, :, pl.ds(c * CHUNK, CHUNK)])
        return carry

    lax.fori_loop(0, BPW * NCHUNK, body, 0)


def kernel(node_feat_idx, degree, edge_feat_idx, adj, position_bias,
           atom_table, edge_table, degree_table, node_vnode,
           node_vnode_distance, diffusion_weight):
    combined = jnp.concatenate([atom_table, degree_table], axis=0)
    idx_all = jnp.concatenate(
        [node_feat_idx, degree[..., None] + ATOM_VOCAB], axis=-1)
    idx_node = idx_all.astype(jnp.int32).reshape(NW, NCHN, ROWS)
    node_features = _node_gather(combined, idx_node).reshape(B, N, D)

    pat, explored = _apow_call(adj)
    pat2 = pat.reshape(B, NPOS)
    bits = ((jnp.arange(NPAT)[:, None] >> jnp.arange(NUM_HOPS)[None, :])
            & 1).astype(jnp.float32)
    faclut = 1.0 + bits @ diffusion_weight
    eidx4 = edge_feat_idx.astype(jnp.int32).reshape(
        B, NCHUNK, CHUNK, 3).transpose(0, 1, 3, 2)
    pos3 = position_bias.reshape(B, H, NPOS)
    att3 = _edge_merge(edge_table, eidx4, pos3, pat2, faclut)
    att_bias = att3.reshape(B, H, N, N)
    return (node_features, att_bias, explored, node_vnode,
            node_vnode_distance)


# speedup vs baseline: 1.6985x; 1.0126x over previous
import functools

import jax
import jax.numpy as jnp
from jax import lax
from jax.experimental import pallas as pl
from jax.experimental.pallas import tpu as pltpu
from jax.experimental.pallas import tpu_sc as plsc

B, N, H, D = 128, 64, 32, 768
NUM_HOPS = 5
ATOM_VOCAB = 512 * 9 + 1
EDGE_VOCAB = 512 * 3 + 1
DEG_VOCAB = 512
NPAT = 1 << NUM_HOPS

NPOS = N * N
NCHUNK = 8
CHUNK = NPOS // NCHUNK

NC, NS = 2, 16
NW = NC * NS

BB = 8


def _apow_body(adj_ref, pat_ref, explored_ref):
    a32 = adj_ref[...]
    a16 = a32.astype(jnp.bfloat16)
    ak = a16
    acc = a32
    pat = jnp.zeros_like(a32)
    for hop in range(NUM_HOPS):
        pat = pat + float(1 << hop) * ak.astype(jnp.float32)
        prod = lax.dot_general(
            ak, a16,
            dimension_numbers=(((2,), (1,)), ((0,), (0,))),
            preferred_element_type=jnp.float32)
        akn = jnp.minimum(prod, 1.0)
        acc = acc + akn
        ak = akn.astype(jnp.bfloat16)
    pat_ref[...] = pat.astype(jnp.int32)
    explored_ref[...] = (acc > 0).astype(jnp.float32)


_apow_call = pl.pallas_call(
    _apow_body,
    grid=(B // BB,),
    in_specs=[pl.BlockSpec((BB, N, N), lambda i: (i, 0, 0))],
    out_specs=[
        pl.BlockSpec((BB, N, N), lambda i: (i, 0, 0)),
        pl.BlockSpec((BB, N, N), lambda i: (i, 0, 0)),
    ],
    out_shape=[
        jax.ShapeDtypeStruct((B, N, N), jnp.int32),
        jax.ShapeDtypeStruct((B, N, N), jnp.float32),
    ],
)

PAIRS = B * N
PPW = PAIRS // NW
CP = 4
ROWS = CP * 10
NCHN = PPW // CP

_sc_mesh = plsc.VectorSubcoreMesh(core_axis_name="c", subcore_axis_name="s")


@functools.partial(
    pl.kernel,
    mesh=_sc_mesh,
    out_type=jax.ShapeDtypeStruct((PAIRS, D), jnp.float32),
    compiler_params=pltpu.CompilerParams(
        needs_layout_passes=False, use_tc_tiling_on_sc=False),
    scratch_types=[
        pltpu.VMEM((NCHN, ROWS), jnp.int32),
        pltpu.VMEM((ROWS, D), jnp.float32),
        pltpu.VMEM((CP, D), jnp.float32),
        pltpu.SemaphoreType.DMA,
    ],
)
def _node_gather(table_hbm, idx_hbm, out_hbm, idx_v, rows_v, out_v, sem):
    wid = lax.axis_index("s") * NC + lax.axis_index("c")
    pltpu.sync_copy(idx_hbm.at[wid], idx_v)

    def chunk(c, carry):
        pltpu.async_copy(table_hbm.at[idx_v.at[c]], rows_v, sem).wait()
        for p in range(CP):
            def dloop(j, carry2):
                sl = pl.ds(j * 16, 16)
                acc = rows_v[p * 10, sl]
                for k in range(1, 10):
                    acc = acc + rows_v[p * 10 + k, sl]
                out_v[p, sl] = acc
                return carry2
            lax.fori_loop(0, D // 16, dloop, 0)
        pltpu.sync_copy(out_v, out_hbm.at[pl.ds(wid * PPW + c * CP, CP), :])
        return carry

    lax.fori_loop(0, NCHN, chunk, 0)


BPW = B // NW


@functools.partial(
    pl.kernel,
    mesh=_sc_mesh,
    out_type=jax.ShapeDtypeStruct((B, H, NPOS), jnp.float32),
    compiler_params=pltpu.CompilerParams(
        needs_layout_passes=False, use_tc_tiling_on_sc=False),
    scratch_types=[
        pltpu.VMEM((EDGE_VOCAB, H), jnp.float32),
        pltpu.VMEM((3, CHUNK), jnp.int32),
        pltpu.VMEM((CHUNK,), jnp.int32),
        pltpu.VMEM((H, CHUNK), jnp.float32),
        pltpu.VMEM((H, CHUNK), jnp.float32),
        pltpu.VMEM((NPAT, H), jnp.float32),
        pltpu.SemaphoreType.DMA,
    ],
)
def _edge_merge(tab_hbm, eidx_hbm, pos_hbm, pat_hbm, faclut_hbm, att_hbm,
                tab_v, eidx_v, pat_v, pos_v, out_v, faclut_v, sem):
    wid = lax.axis_index("s") * NC + lax.axis_index("c")
    pltpu.sync_copy(tab_hbm, tab_v)
    pltpu.sync_copy(faclut_hbm, faclut_v)

    def body(t, carry):
        b = wid * BPW + t // NCHUNK
        c = t % NCHUNK
        pltpu.sync_copy(eidx_hbm.at[b, c], eidx_v)
        pltpu.sync_copy(pat_hbm.at[b, pl.ds(c * CHUNK, CHUNK)], pat_v)
        pltpu.sync_copy(pos_hbm.at[b, :, pl.ds(c * CHUNK, CHUNK)], pos_v)

        def group(g, carry2):
            sl = pl.ds(g * 16, 16)
            e0 = eidx_v[0, sl]
            e1 = eidx_v[1, sl]
            e2 = eidx_v[2, sl]
            pat = pat_v[sl]
            HB = 4

            def loads(hb):
                hs = [jnp.full((16,), hb + i, jnp.int32) for i in range(HB)]
                fs = [plsc.load_gather(faclut_v, [pat, hs[i]])
                      for i in range(HB)]
                p_s = [pos_v[hb + i, sl] for i in range(HB)]
                g0s = [plsc.load_gather(tab_v, [e0, hs[i]]) for i in range(HB)]
                g1s = [plsc.load_gather(tab_v, [e1, hs[i]]) for i in range(HB)]
                g2s = [plsc.load_gather(tab_v, [e2, hs[i]]) for i in range(HB)]
                return fs, p_s, g0s, g1s, g2s

            cur = loads(0)
            for hb in range(0, H, HB):
                nxt = loads(hb + HB) if hb + HB < H else None
                fs, p_s, g0s, g1s, g2s = cur
                for i in range(HB):
                    out_v[hb + i, sl] = (
                        (p_s[i] + g0s[i] + g1s[i] + g2s[i]) * fs[i])
                cur = nxt
            return carry2

        lax.fori_loop(0, CHUNK // 16, group, 0)

        pltpu.sync_copy(out_v, att_hbm.at[b, :, pl.ds(c * CHUNK, CHUNK)])
        return carry

    lax.fori_loop(0, BPW * NCHUNK, body, 0)


def kernel(node_feat_idx, degree, edge_feat_idx, adj, position_bias,
           atom_table, edge_table, degree_table, node_vnode,
           node_vnode_distance, diffusion_weight):
    combined = jnp.concatenate([atom_table, degree_table], axis=0)
    idx_all = jnp.concatenate(
        [node_feat_idx, degree[..., None] + ATOM_VOCAB], axis=-1)
    idx_node = idx_all.astype(jnp.int32).reshape(NW, NCHN, ROWS)
    node_features = _node_gather(combined, idx_node).reshape(B, N, D)

    pat, explored = _apow_call(adj)
    pat2 = pat.reshape(B, NPOS)
    bits = ((jnp.arange(NPAT)[:, None] >> jnp.arange(NUM_HOPS)[None, :])
            & 1).astype(jnp.float32)
    faclut = 1.0 + bits @ diffusion_weight
    eidx4 = edge_feat_idx.astype(jnp.int32).reshape(
        B, NCHUNK, CHUNK, 3).transpose(0, 1, 3, 2)
    pos3 = position_bias.reshape(B, H, NPOS)
    att3 = _edge_merge(edge_table, eidx4, pos3, pat2, faclut)
    att_bias = att3.reshape(B, H, N, N)
    return (node_features, att_bias, explored, node_vnode,
            node_vnode_distance)


# speedup vs baseline: 1.7794x; 1.0476x over previous
import functools

import jax
import jax.numpy as jnp
from jax import lax
from jax.experimental import pallas as pl
from jax.experimental.pallas import tpu as pltpu
from jax.experimental.pallas import tpu_sc as plsc

B, N, H, D = 128, 64, 32, 768
NUM_HOPS = 5
ATOM_VOCAB = 512 * 9 + 1
EDGE_VOCAB = 512 * 3 + 1
DEG_VOCAB = 512
NPAT = 1 << NUM_HOPS

NPOS = N * N
NCHUNK = 8
CHUNK = NPOS // NCHUNK

NC, NS = 2, 16
NW = NC * NS

BB = 8


def _apow_body(adj_ref, pat_ref, explored_ref):
    a32 = adj_ref[...]
    a16 = a32.astype(jnp.bfloat16)
    ak = a16
    acc = a32
    pat = jnp.zeros_like(a32)
    for hop in range(NUM_HOPS):
        pat = pat + float(1 << hop) * ak.astype(jnp.float32)
        prod = lax.dot_general(
            ak, a16,
            dimension_numbers=(((2,), (1,)), ((0,), (0,))),
            preferred_element_type=jnp.float32)
        akn = jnp.minimum(prod, 1.0)
        acc = acc + akn
        ak = akn.astype(jnp.bfloat16)
    pat_ref[...] = pat.astype(jnp.int32)
    explored_ref[...] = (acc > 0).astype(jnp.float32)


_apow_call = pl.pallas_call(
    _apow_body,
    grid=(B // BB,),
    in_specs=[pl.BlockSpec((BB, N, N), lambda i: (i, 0, 0))],
    out_specs=[
        pl.BlockSpec((BB, N, N), lambda i: (i, 0, 0)),
        pl.BlockSpec((BB, N, N), lambda i: (i, 0, 0)),
    ],
    out_shape=[
        jax.ShapeDtypeStruct((B, N, N), jnp.int32),
        jax.ShapeDtypeStruct((B, N, N), jnp.float32),
    ],
)

PAIRS = B * N
PPW = PAIRS // NW
CP = 4
AROWS = CP * 9
NCHN = PPW // CP

_sc_mesh = plsc.VectorSubcoreMesh(core_axis_name="c", subcore_axis_name="s")


@functools.partial(
    pl.kernel,
    mesh=_sc_mesh,
    out_type=jax.ShapeDtypeStruct((PAIRS, D), jnp.float32),
    compiler_params=pltpu.CompilerParams(
        needs_layout_passes=False, use_tc_tiling_on_sc=False,
        disable_bounds_checks=True),
    scratch_types=[
        pltpu.VMEM((NCHN, AROWS), jnp.int32),
        pltpu.VMEM((NCHN, CP), jnp.int32),
        pltpu.VMEM((2, AROWS, D), jnp.float32),
        pltpu.VMEM((2, CP, D), jnp.float32),
        pltpu.VMEM((CP, D), jnp.float32),
        pltpu.SemaphoreType.DMA,
        pltpu.SemaphoreType.DMA,
    ],
)
def _node_gather(atab_hbm, dtab_hbm, aidx_hbm, didx_hbm, out_hbm,
                 aidx_v, didx_v, arows_v, drows_v, out_v, sem0, sem1):
    wid = lax.axis_index("s") * NC + lax.axis_index("c")
    pltpu.sync_copy(aidx_hbm.at[wid], aidx_v)
    pltpu.sync_copy(didx_hbm.at[wid], didx_v)
    sems = (sem0, sem1)

    def fire(c, buf):
        pltpu.async_copy(atab_hbm.at[aidx_v.at[c]], arows_v.at[buf],
                         sems[buf])
        pltpu.async_copy(dtab_hbm.at[didx_v.at[c]], drows_v.at[buf],
                         sems[buf])

    fire(0, 0)

    def chunk2(cc, carry):
        for buf in range(2):
            c = cc * 2 + buf
            pltpu.make_async_copy(atab_hbm.at[aidx_v.at[c]],
                                  arows_v.at[buf], sems[buf]).wait()
            pltpu.make_async_copy(dtab_hbm.at[didx_v.at[c]],
                                  drows_v.at[buf], sems[buf]).wait()

            @pl.when(c + 1 < NCHN)
            def _():
                fire(c + 1, 1 - buf)

            for p in range(CP):
                def dloop(j, carry2):
                    sl = pl.ds(j * 16, 16)
                    acc = drows_v[buf, p, sl]
                    for k in range(9):
                        acc = acc + arows_v[buf, p * 9 + k, sl]
                    out_v[p, sl] = acc
                    return carry2
                lax.fori_loop(0, D // 16, dloop, 0)
            pltpu.sync_copy(
                out_v, out_hbm.at[pl.ds(wid * PPW + c * CP, CP), :])
        return carry

    lax.fori_loop(0, NCHN // 2, chunk2, 0)


BPW = B // NW


@functools.partial(
    pl.kernel,
    mesh=_sc_mesh,
    out_type=jax.ShapeDtypeStruct((B, H, NPOS), jnp.float32),
    compiler_params=pltpu.CompilerParams(
        needs_layout_passes=False, use_tc_tiling_on_sc=False,
        disable_bounds_checks=True),
    scratch_types=[
        pltpu.VMEM((EDGE_VOCAB, H), jnp.float32),
        pltpu.VMEM((3, CHUNK), jnp.int32),
        pltpu.VMEM((CHUNK,), jnp.int32),
        pltpu.VMEM((H, CHUNK), jnp.float32),
        pltpu.VMEM((H, CHUNK), jnp.float32),
        pltpu.VMEM((NPAT, H), jnp.float32),
        pltpu.SemaphoreType.DMA,
    ],
)
def _edge_merge(tab_hbm, eidx_hbm, pos_hbm, pat_hbm, faclut_hbm, att_hbm,
                tab_v, eidx_v, pat_v, pos_v, out_v, faclut_v, sem):
    wid = lax.axis_index("s") * NC + lax.axis_index("c")
    pltpu.sync_copy(tab_hbm, tab_v)
    pltpu.sync_copy(faclut_hbm, faclut_v)

    def body(t, carry):
        b = wid * BPW + t // NCHUNK
        c = t % NCHUNK
        pltpu.sync_copy(eidx_hbm.at[b, c], eidx_v)
        pltpu.sync_copy(pat_hbm.at[b, pl.ds(c * CHUNK, CHUNK)], pat_v)
        pltpu.sync_copy(pos_hbm.at[b, :, pl.ds(c * CHUNK, CHUNK)], pos_v)

        def group(g, carry2):
            sl = pl.ds(g * 16, 16)
            e0 = eidx_v[0, sl]
            e1 = eidx_v[1, sl]
            e2 = eidx_v[2, sl]
            pat = pat_v[sl]
            HB = 4

            def loads(hb):
                hs = [jnp.full((16,), hb + i, jnp.int32) for i in range(HB)]
                fs = [plsc.load_gather(faclut_v, [pat, hs[i]])
                      for i in range(HB)]
                p_s = [pos_v[hb + i, sl] for i in range(HB)]
                g0s = [plsc.load_gather(tab_v, [e0, hs[i]]) for i in range(HB)]
                g1s = [plsc.load_gather(tab_v, [e1, hs[i]]) for i in range(HB)]
                g2s = [plsc.load_gather(tab_v, [e2, hs[i]]) for i in range(HB)]
                return fs, p_s, g0s, g1s, g2s

            cur = loads(0)
            for hb in range(0, H, HB):
                nxt = loads(hb + HB) if hb + HB < H else None
                fs, p_s, g0s, g1s, g2s = cur
                for i in range(HB):
                    out_v[hb + i, sl] = (
                        (p_s[i] + g0s[i] + g1s[i] + g2s[i]) * fs[i])
                cur = nxt
            return carry2

        lax.fori_loop(0, CHUNK // 16, group, 0)

        pltpu.sync_copy(out_v, att_hbm.at[b, :, pl.ds(c * CHUNK, CHUNK)])
        return carry

    lax.fori_loop(0, BPW * NCHUNK, body, 0)


def kernel(node_feat_idx, degree, edge_feat_idx, adj, position_bias,
           atom_table, edge_table, degree_table, node_vnode,
           node_vnode_distance, diffusion_weight):
    aidx = node_feat_idx.astype(jnp.int32).reshape(NW, NCHN, AROWS)
    didx = degree.astype(jnp.int32).reshape(NW, NCHN, CP)
    node_features = _node_gather(
        atom_table, degree_table, aidx, didx).reshape(B, N, D)

    pat, explored = _apow_call(adj)
    pat2 = pat.reshape(B, NPOS)
    bits = ((jnp.arange(NPAT)[:, None] >> jnp.arange(NUM_HOPS)[None, :])
            & 1).astype(jnp.float32)
    faclut = 1.0 + bits @ diffusion_weight
    eidx4 = edge_feat_idx.astype(jnp.int32).reshape(
        B, NCHUNK, CHUNK, 3).transpose(0, 1, 3, 2)
    pos3 = position_bias.reshape(B, H, NPOS)
    att3 = _edge_merge(edge_table, eidx4, pos3, pat2, faclut)
    att_bias = att3.reshape(B, H, N, N)
    return (node_features, att_bias, explored, node_vnode,
            node_vnode_distance)
